# Initial kernel scaffold; baseline (speedup 1.0000x reference)
import jax
import jax.numpy as jnp
from jax.experimental import pallas as pl


def kernel(node_feats, edge_feats, edge_index, proj_W, proj_b, W1, b1, W2, b2, nn_bias, Wb, bb, gamma, beta):
    raise NotImplementedError("write your pallas kernel here")


# speedup vs baseline: 1.1358x; 1.1358x over previous
import functools

import jax
import jax.numpy as jnp
import numpy as np
from jax.experimental import pallas as pl
from jax.experimental.pallas import tpu as pltpu

N = 10000
E = 160000
DIN = 128
DE = 16
DOUT = 32
DH = 128
STEPS = 6

EB_W = 2000
EB_M = 1000
EB_L = 2000


def _h0_body(x_ref, w_ref, b_ref, o_ref):
    o_ref[...] = jax.nn.relu(
        jnp.dot(x_ref[...], w_ref[...], preferred_element_type=jnp.float32)
        + b_ref[...]
    )


def _h0(node_feats, proj_W, proj_b):
    return pl.pallas_call(
        _h0_body,
        out_shape=jax.ShapeDtypeStruct((N, DOUT), jnp.float32),
    )(node_feats, proj_W, proj_b.reshape(1, DOUT))


def _we_body(ef_ref, w1_ref, b1_ref, w2_ref, b2_ref, o_ref):
    z = jax.nn.relu(
        jnp.dot(ef_ref[...], w1_ref[...], preferred_element_type=jnp.float32)
        + b1_ref[...]
    )
    we = jnp.dot(z, w2_ref[...], preferred_element_type=jnp.float32) + b2_ref[...]
    o_ref[...] = we.astype(jnp.bfloat16)


def _we(edge_feats, W1, b1, W2, b2):
    grid = E // EB_W
    return pl.pallas_call(
        _we_body,
        grid=(grid,),
        in_specs=[
            pl.BlockSpec((EB_W, DE), lambda i: (i, 0)),
            pl.BlockSpec((DE, DH), lambda i: (0, 0)),
            pl.BlockSpec((1, DH), lambda i: (0, 0)),
            pl.BlockSpec((DH, DOUT * DOUT), lambda i: (0, 0)),
            pl.BlockSpec((1, DOUT * DOUT), lambda i: (0, 0)),
        ],
        out_specs=pl.BlockSpec((EB_W, DOUT * DOUT), lambda i: (i, 0)),
        out_shape=jax.ShapeDtypeStruct((E, DOUT * DOUT), jnp.bfloat16),
    )(edge_feats, W1, b1.reshape(1, DH), W2, b2.reshape(1, DOUT * DOUT))


def _msg_body(h_ref, we_ref, r_ref, o_ref):
    h = h_ref[...]
    s = jnp.zeros((EB_M, 128), jnp.float32)
    for c in range(8):
        hE = jnp.dot(h, r_ref[:, c * 128:(c + 1) * 128],
                     preferred_element_type=jnp.float32)
        w = we_ref[:, c * 128:(c + 1) * 128].astype(jnp.float32)
        s = s + w * hE
    m = (s[:, 0:32] + s[:, 32:64]) + (s[:, 64:96] + s[:, 96:128])
    o_ref[...] = m


def _messages(h_src, we):
    grid = E // EB_M
    r = np.zeros((DOUT, DOUT * DOUT), np.float32)
    for i in range(DOUT):
        r[i, i * DOUT:(i + 1) * DOUT] = 1.0
    r = jnp.asarray(r)
    return pl.pallas_call(
        _msg_body,
        grid=(grid,),
        in_specs=[
            pl.BlockSpec((EB_M, DOUT), lambda i: (i, 0)),
            pl.BlockSpec((EB_M, DOUT * DOUT), lambda i: (i, 0)),
            pl.BlockSpec((DOUT, DOUT * DOUT), lambda i: (0, 0)),
        ],
        out_specs=pl.BlockSpec((EB_M, DOUT), lambda i: (i, 0)),
        out_shape=jax.ShapeDtypeStruct((E, DOUT), jnp.float32),
    )(h_src, we, r)


def _p_body(h_ref, w_ref, o_ref):
    o_ref[...] = jnp.dot(h_ref[...], w_ref[...],
                         preferred_element_type=jnp.float32)


def _pmat(h, Wb):
    wcat = jnp.concatenate([Wb[:DOUT], Wb[DOUT:]], axis=1)
    return pl.pallas_call(
        _p_body,
        out_shape=jax.ShapeDtypeStruct((N, 2 * DOUT), jnp.float32),
    )(h, wcat)


def _ln_body(x_ref, g_ref, b_ref, o_ref):
    x = x_ref[...]
    mu = jnp.mean(x, axis=-1, keepdims=True)
    var = jnp.mean((x - mu) ** 2, axis=-1, keepdims=True)
    o_ref[...] = (x - mu) / jnp.sqrt(var + 1e-5) * g_ref[...] + b_ref[...]


def _ln_atom(x, gamma, beta):
    return pl.pallas_call(
        _ln_body,
        out_shape=jax.ShapeDtypeStruct((N, DOUT), jnp.float32),
    )(x, gamma.reshape(1, DOUT), beta.reshape(1, DOUT))


def _ln_bond(x, gamma, beta):
    grid = E // EB_L
    return pl.pallas_call(
        _ln_body,
        grid=(grid,),
        in_specs=[
            pl.BlockSpec((EB_L, DOUT), lambda i: (i, 0)),
            pl.BlockSpec((1, DOUT), lambda i: (0, 0)),
            pl.BlockSpec((1, DOUT), lambda i: (0, 0)),
        ],
        out_specs=pl.BlockSpec((EB_L, DOUT), lambda i: (i, 0)),
        out_shape=jax.ShapeDtypeStruct((E, DOUT), jnp.float32),
    )(x, gamma.reshape(1, DOUT), beta.reshape(1, DOUT))


def kernel(node_feats, edge_feats, edge_index, proj_W, proj_b, W1, b1, W2, b2,
           nn_bias, Wb, bb, gamma, beta):
    src = edge_index[0]
    dst = edge_index[1]

    h = _h0(node_feats, proj_W, proj_b)
    we = _we(edge_feats, W1, b1, W2, b2)

    for _ in range(STEPS):
        h_src = jnp.take(h, src, axis=0)
        m = _messages(h_src, we)
        agg = jax.ops.segment_sum(m, dst, num_segments=N)
        h = jax.nn.relu(agg + nn_bias)

    p = _pmat(h, Wb)
    bond_pre = jnp.take(p[:, :DOUT], src, axis=0) + jnp.take(p[:, DOUT:], dst, axis=0) + bb
    atom = _ln_atom(h, gamma, beta)
    bond = _ln_bond(bond_pre, gamma, beta)
    return (atom, bond)


# speedup vs baseline: 2.2895x; 2.0158x over previous
import functools

import jax
import jax.numpy as jnp
import numpy as np
from jax import lax
from jax.experimental import pallas as pl
from jax.experimental.pallas import tpu as pltpu
from jax.experimental.pallas import tpu_sc as plsc

N = 10000
E = 160000
DIN = 128
DE = 16
DOUT = 32
DH = 128
STEPS = 6
TW = 128

NC = 2
NS = 16
NW = NC * NS
EPW = E // NW
CH = 200
NCH = N // CH
NKR = -(-NCH // NS)
ZCH = 80
NZC = N // ZCH
NZR = -(-NZC // NS)

EB_W = 2000
EB_M = 1000
EB_L = 2000

_MESH = plsc.VectorSubcoreMesh(core_axis_name="c", subcore_axis_name="s")


def _gather_body(tab, idxa, idxb, outa, outb,
                 idx_va, idx_vb, rows_va, rows_vb, sem0, sem1):
    wid = lax.axis_index("s") * NC + lax.axis_index("c")
    base = wid * EPW

    def body(j, carry):
        b = base + j * CH
        pltpu.sync_copy(idxa.at[pl.ds(b, CH)], idx_va)
        pltpu.sync_copy(idxb.at[pl.ds(b, CH)], idx_vb)
        cpa = pltpu.async_copy(tab.at[idx_va], rows_va, sem0)
        cpb = pltpu.async_copy(tab.at[idx_vb], rows_vb, sem1)
        cpa.wait()
        cpb.wait()
        pltpu.sync_copy(rows_va, outa.at[pl.ds(b, CH)])
        pltpu.sync_copy(rows_vb, outb.at[pl.ds(b, CH)])
        return carry

    lax.fori_loop(0, EPW // CH, body, 0)


@functools.partial(
    pl.kernel,
    out_type=[jax.ShapeDtypeStruct((E, TW), jnp.float32),
              jax.ShapeDtypeStruct((E, TW), jnp.float32)],
    mesh=_MESH,
    scratch_types=[
        pltpu.VMEM((CH,), jnp.int32),
        pltpu.VMEM((CH,), jnp.int32),
        pltpu.VMEM((CH, TW), jnp.float32),
        pltpu.VMEM((CH, TW), jnp.float32),
        pltpu.SemaphoreType.DMA,
        pltpu.SemaphoreType.DMA,
    ],
)
def _gather(tab, idxa, idxb, outa, outb,
            idx_va, idx_vb, rows_va, rows_vb, sem0, sem1):
    _gather_body(tab, idxa, idxb, outa, outb,
                 idx_va, idx_vb, rows_va, rows_vb, sem0, sem1)


def _scatter_body(m, dst, pcat, idx_v, rows_v, zbuf_v, agg_sh, sem):
    cid = lax.axis_index("c")
    sid = lax.axis_index("s")
    wid = sid * NC + cid
    base = wid * EPW

    def zrow(r, carry):
        for c16 in range(TW // 16):
            zbuf_v[r, pl.ds(c16 * 16, 16)] = jnp.zeros((16,), jnp.float32)
        return carry

    lax.fori_loop(0, ZCH, zrow, 0)

    def zchunk(k, carry):
        c = sid + k * NS

        @pl.when(c < NZC)
        def _():
            pltpu.sync_copy(zbuf_v, agg_sh.at[pl.ds(c * ZCH, ZCH)])

        return carry

    lax.fori_loop(0, NZR, zchunk, 0)
    plsc.subcore_barrier()

    def body(j, carry):
        b = base + j * CH
        pltpu.sync_copy(dst.at[pl.ds(b, CH)], idx_v)
        pltpu.sync_copy(m.at[pl.ds(b, CH)], rows_v)
        pltpu.sync_copy(rows_v, agg_sh.at[idx_v], add=True)
        return carry

    lax.fori_loop(0, EPW // CH, body, 0)
    plsc.subcore_barrier()

    def wchunk(k, carry):
        c = sid + k * NS

        @pl.when(c < NCH)
        def _():
            pltpu.sync_copy(agg_sh.at[pl.ds(c * CH, CH)],
                            pcat.at[pl.ds(cid * N + c * CH, CH)])

        return carry

    lax.fori_loop(0, NKR, wchunk, 0)


@functools.partial(
    pl.kernel,
    out_type=jax.ShapeDtypeStruct((2 * N, TW), jnp.float32),
    mesh=_MESH,
    scratch_types=[
        pltpu.VMEM((CH,), jnp.int32),
        pltpu.VMEM((CH, TW), jnp.float32),
        pltpu.VMEM((ZCH, TW), jnp.float32),
        pltpu.VMEM_SHARED((N, TW), jnp.float32),
        pltpu.SemaphoreType.DMA,
    ],
)
def _scatter(m, dst, pcat, idx_v, rows_v, zbuf_v, agg_sh, sem):
    _scatter_body(m, dst, pcat, idx_v, rows_v, zbuf_v, agg_sh, sem)


def _h0_body(x_ref, w_ref, b_ref, o_ref):
    h = (jnp.dot(x_ref[...], w_ref[...], preferred_element_type=jnp.float32)
         + b_ref[...])
    pad = jnp.concatenate([h, jnp.zeros((N, TW - DOUT), jnp.float32)], axis=1)
    o_ref[...] = jnp.concatenate([pad, jnp.zeros((N, TW), jnp.float32)],
                                 axis=0)


def _h0(node_feats, proj_W, proj_b, nn_bias):
    return pl.pallas_call(
        _h0_body,
        out_shape=jax.ShapeDtypeStruct((2 * N, TW), jnp.float32),
    )(node_feats, proj_W, (proj_b - nn_bias).reshape(1, DOUT))


def _we_body(ef_ref, w1_ref, b1_ref, w2_ref, b2_ref, o_ref):
    z = jax.nn.relu(
        jnp.dot(ef_ref[...], w1_ref[...], preferred_element_type=jnp.float32)
        + b1_ref[...]
    )
    we = jnp.dot(z, w2_ref[...], preferred_element_type=jnp.float32) + b2_ref[...]
    o_ref[...] = we.astype(jnp.bfloat16)


def _we(edge_feats, W1, b1, W2, b2):
    grid = E // EB_W
    return pl.pallas_call(
        _we_body,
        grid=(grid,),
        in_specs=[
            pl.BlockSpec((EB_W, DE), lambda i: (i, 0)),
            pl.BlockSpec((DE, DH), lambda i: (0, 0)),
            pl.BlockSpec((1, DH), lambda i: (0, 0)),
            pl.BlockSpec((DH, DOUT * DOUT), lambda i: (0, 0)),
            pl.BlockSpec((1, DOUT * DOUT), lambda i: (0, 0)),
        ],
        out_specs=pl.BlockSpec((EB_W, DOUT * DOUT), lambda i: (i, 0)),
        out_shape=jax.ShapeDtypeStruct((E, DOUT * DOUT), jnp.bfloat16),
    )(edge_feats, W1, b1.reshape(1, DH), W2, b2.reshape(1, DOUT * DOUT))


def _msg_body(hs0_ref, hs1_ref, b_ref, we_ref, r_ref, o_ref):
    h = jax.nn.relu(hs0_ref[:, 0:DOUT] + hs1_ref[:, 0:DOUT] + b_ref[...])
    s = jnp.zeros((EB_M, 128), jnp.float32)
    for c in range(8):
        hE = jnp.dot(h, r_ref[:, c * 128:(c + 1) * 128],
                     preferred_element_type=jnp.float32)
        w = we_ref[:, c * 128:(c + 1) * 128].astype(jnp.float32)
        s = s + w * hE
    m = (s[:, 0:32] + s[:, 32:64]) + (s[:, 64:96] + s[:, 96:128])
    o_ref[...] = jnp.concatenate(
        [m, jnp.zeros((EB_M, TW - DOUT), jnp.float32)], axis=1)


def _r_mat():
    r = np.zeros((DOUT, DOUT * DOUT), np.float32)
    for i in range(DOUT):
        r[i, i * DOUT:(i + 1) * DOUT] = 1.0
    return jnp.asarray(r)


def _messages(hs0, hs1, nn_bias, we):
    grid = E // EB_M
    return pl.pallas_call(
        _msg_body,
        grid=(grid,),
        in_specs=[
            pl.BlockSpec((EB_M, TW), lambda i: (i, 0)),
            pl.BlockSpec((EB_M, TW), lambda i: (i, 0)),
            pl.BlockSpec((1, DOUT), lambda i: (0, 0)),
            pl.BlockSpec((EB_M, DOUT * DOUT), lambda i: (i, 0)),
            pl.BlockSpec((DOUT, DOUT * DOUT), lambda i: (0, 0)),
        ],
        out_specs=pl.BlockSpec((EB_M, TW), lambda i: (i, 0)),
        out_shape=jax.ShapeDtypeStruct((E, TW), jnp.float32),
    )(hs0, hs1, nn_bias.reshape(1, DOUT), we, _r_mat())


def _final_body(pc_ref, b_ref, w_ref, g_ref, bb_ref, atom_ref, p_ref):
    h = jax.nn.relu(pc_ref[0:N, 0:DOUT] + pc_ref[N:2 * N, 0:DOUT] + b_ref[...])
    mu = jnp.mean(h, axis=-1, keepdims=True)
    var = jnp.mean((h - mu) ** 2, axis=-1, keepdims=True)
    atom_ref[...] = (h - mu) / jnp.sqrt(var + 1e-5) * g_ref[...] + bb_ref[...]
    p = jnp.dot(h, w_ref[...], preferred_element_type=jnp.float32)
    zpad = jnp.zeros((N, TW - 2 * DOUT), jnp.float32)
    top = jnp.concatenate([p[:, 0:DOUT], jnp.zeros((N, DOUT), jnp.float32),
                           zpad], axis=1)
    bot = jnp.concatenate([jnp.zeros((N, DOUT), jnp.float32), p[:, DOUT:],
                           zpad], axis=1)
    p_ref[...] = jnp.concatenate([top, bot], axis=0)


def _final(pcat, nn_bias, Wb, gamma, beta):
    wcat = jnp.concatenate([Wb[:DOUT], Wb[DOUT:]], axis=1)
    return pl.pallas_call(
        _final_body,
        out_shape=[jax.ShapeDtypeStruct((N, DOUT), jnp.float32),
                   jax.ShapeDtypeStruct((2 * N, TW), jnp.float32)],
    )(pcat, nn_bias.reshape(1, DOUT), wcat,
      gamma.reshape(1, DOUT), beta.reshape(1, DOUT))


def _bond_body(x_ref, y_ref, bb_ref, g_ref, b_ref, o_ref):
    x = x_ref[:, 0:DOUT] + y_ref[:, DOUT:2 * DOUT] + bb_ref[...]
    mu = jnp.mean(x, axis=-1, keepdims=True)
    var = jnp.mean((x - mu) ** 2, axis=-1, keepdims=True)
    o_ref[...] = (x - mu) / jnp.sqrt(var + 1e-5) * g_ref[...] + b_ref[...]


def _bond(bpre0, bpre1, bb, gamma, beta):
    grid = E // EB_L
    return pl.pallas_call(
        _bond_body,
        grid=(grid,),
        in_specs=[
            pl.BlockSpec((EB_L, TW), lambda i: (i, 0)),
            pl.BlockSpec((EB_L, TW), lambda i: (i, 0)),
            pl.BlockSpec((1, DOUT), lambda i: (0, 0)),
            pl.BlockSpec((1, DOUT), lambda i: (0, 0)),
            pl.BlockSpec((1, DOUT), lambda i: (0, 0)),
        ],
        out_specs=pl.BlockSpec((EB_L, DOUT), lambda i: (i, 0)),
        out_shape=jax.ShapeDtypeStruct((E, DOUT), jnp.float32),
    )(bpre0, bpre1, bb.reshape(1, DOUT), gamma.reshape(1, DOUT),
      beta.reshape(1, DOUT))


def kernel(node_feats, edge_feats, edge_index, proj_W, proj_b, W1, b1, W2, b2,
           nn_bias, Wb, bb, gamma, beta):
    src = edge_index[0]
    dst = edge_index[1]
    srcb = src + N
    dstb = dst + N
    tab = _h0(node_feats, proj_W, proj_b, nn_bias)
    we = _we(edge_feats, W1, b1, W2, b2)

    for _ in range(STEPS):
        hs0, hs1 = _gather(tab, src, srcb)
        m = _messages(hs0, hs1, nn_bias, we)
        tab = _scatter(m, dst)

    atom, btab = _final(tab, nn_bias, Wb, gamma, beta)
    bpre0, bpre1 = _gather(btab, src, dstb)
    bond = _bond(bpre0, bpre1, bb, gamma, beta)
    return (atom, bond)


# speedup vs baseline: 2.6415x; 1.1538x over previous
import functools

import jax
import jax.numpy as jnp
import numpy as np
from jax import lax
from jax.experimental import pallas as pl
from jax.experimental.pallas import tpu as pltpu
from jax.experimental.pallas import tpu_sc as plsc

N = 10000
E = 160000
E4 = E // 4
DIN = 128
DE = 16
DOUT = 32
DH = 128
STEPS = 6
TW = 128

NC = 2
NS = 16
NW = NC * NS

CHG = 256
CHG4 = CHG // 4
NCHG = E // CHG
GRND = -(-NCHG // NW)

WCH = 200
NWCH = N // WCH
WRND = -(-NWCH // NS)
ZCH = 40
NZC = N // ZCH
NZR = -(-NZC // NS)

EB_W = 1600
EB_M = 1600
EB_L = 2000

_MESH = plsc.VectorSubcoreMesh(core_axis_name="c", subcore_axis_name="s")


def _gather_packed_body(tab, idxa, idxb, outa, outb,
                        idx_va, idx_vb, rows_va, rows_vb, pk_va, pk_vb,
                        sem0, sem1):
    wid = lax.axis_index("s") * NC + lax.axis_index("c")

    def round_(r, carry):
        c = wid + r * NW

        @pl.when(c < NCHG)
        def _():
            b = c * CHG
            pltpu.sync_copy(idxa.at[pl.ds(b, CHG)], idx_va)
            pltpu.sync_copy(idxb.at[pl.ds(b, CHG)], idx_vb)
            cpa = pltpu.async_copy(tab.at[idx_va], rows_va, sem0)
            cpb = pltpu.async_copy(tab.at[idx_vb], rows_vb, sem1)
            cpa.wait()
            cpb.wait()

            def pack(g, carry2):
                for k in range(4):
                    for h in range(2):
                        pk_va[g, pl.ds(k * 32 + h * 16, 16)] = (
                            rows_va[4 * g + k, pl.ds(h * 16, 16)])
                        pk_vb[g, pl.ds(k * 32 + h * 16, 16)] = (
                            rows_vb[4 * g + k, pl.ds(h * 16, 16)])
                return carry2

            lax.fori_loop(0, CHG4, pack, 0)
            pltpu.sync_copy(pk_va, outa.at[pl.ds(c * CHG4, CHG4)])
            pltpu.sync_copy(pk_vb, outb.at[pl.ds(c * CHG4, CHG4)])

        return carry

    lax.fori_loop(0, GRND, round_, 0)


@functools.partial(
    pl.kernel,
    out_type=[jax.ShapeDtypeStruct((E4, TW), jnp.float32),
              jax.ShapeDtypeStruct((E4, TW), jnp.float32)],
    mesh=_MESH,
    scratch_types=[
        pltpu.VMEM((CHG,), jnp.int32),
        pltpu.VMEM((CHG,), jnp.int32),
        pltpu.VMEM((CHG, TW), jnp.float32),
        pltpu.VMEM((CHG, TW), jnp.float32),
        pltpu.VMEM((CHG4, TW), jnp.float32),
        pltpu.VMEM((CHG4, TW), jnp.float32),
        pltpu.SemaphoreType.DMA,
        pltpu.SemaphoreType.DMA,
    ],
)
def _gather_packed(tab, idxa, idxb, outa, outb,
                   idx_va, idx_vb, rows_va, rows_vb, pk_va, pk_vb,
                   sem0, sem1):
    _gather_packed_body(tab, idxa, idxb, outa, outb,
                        idx_va, idx_vb, rows_va, rows_vb, pk_va, pk_vb,
                        sem0, sem1)


def _gather_wide_body(tab, idxa, idxb, outa, outb,
                      idx_va, idx_vb, rows_va, rows_vb, sem0, sem1):
    wid = lax.axis_index("s") * NC + lax.axis_index("c")

    def round_(r, carry):
        c = wid + r * NW

        @pl.when(c < NCHG)
        def _():
            b = c * CHG
            pltpu.sync_copy(idxa.at[pl.ds(b, CHG)], idx_va)
            pltpu.sync_copy(idxb.at[pl.ds(b, CHG)], idx_vb)
            cpa = pltpu.async_copy(tab.at[idx_va], rows_va, sem0)
            cpb = pltpu.async_copy(tab.at[idx_vb], rows_vb, sem1)
            cpa.wait()
            cpb.wait()
            pltpu.sync_copy(rows_va, outa.at[pl.ds(b, CHG)])
            pltpu.sync_copy(rows_vb, outb.at[pl.ds(b, CHG)])

        return carry

    lax.fori_loop(0, GRND, round_, 0)


@functools.partial(
    pl.kernel,
    out_type=[jax.ShapeDtypeStruct((E, TW), jnp.float32),
              jax.ShapeDtypeStruct((E, TW), jnp.float32)],
    mesh=_MESH,
    scratch_types=[
        pltpu.VMEM((CHG,), jnp.int32),
        pltpu.VMEM((CHG,), jnp.int32),
        pltpu.VMEM((CHG, TW), jnp.float32),
        pltpu.VMEM((CHG, TW), jnp.float32),
        pltpu.SemaphoreType.DMA,
        pltpu.SemaphoreType.DMA,
    ],
)
def _gather_wide(tab, idxa, idxb, outa, outb,
                 idx_va, idx_vb, rows_va, rows_vb, sem0, sem1):
    _gather_wide_body(tab, idxa, idxb, outa, outb,
                      idx_va, idx_vb, rows_va, rows_vb, sem0, sem1)


def _scatter_body(mp, dst, pcat, idx_v, mrows_v, u_v, zbuf_v, agg_sh, sem):
    cid = lax.axis_index("c")
    sid = lax.axis_index("s")
    wid = sid * NC + cid

    def zrow(r, carry):
        for c16 in range(TW // 16):
            zbuf_v[r, pl.ds(c16 * 16, 16)] = jnp.zeros((16,), jnp.float32)
        return carry

    lax.fori_loop(0, ZCH, zrow, 0)

    def zchunk(k, carry):
        c = sid + k * NS

        @pl.when(c < NZC)
        def _():
            pltpu.sync_copy(zbuf_v, agg_sh.at[pl.ds(c * ZCH, ZCH)])

        return carry

    lax.fori_loop(0, NZR, zchunk, 0)
    plsc.subcore_barrier()

    def round_(r, carry):
        c = wid + r * NW

        @pl.when(c < NCHG)
        def _():
            pltpu.sync_copy(dst.at[pl.ds(c * CHG, CHG)], idx_v)
            pltpu.sync_copy(mp.at[pl.ds(c * CHG4, CHG4)], mrows_v)

            def unpack(g, carry2):
                for k in range(4):
                    for h in range(2):
                        u_v[4 * g + k, pl.ds(h * 16, 16)] = (
                            mrows_v[g, pl.ds(k * 32 + h * 16, 16)])
                return carry2

            lax.fori_loop(0, CHG4, unpack, 0)
            pltpu.sync_copy(u_v, agg_sh.at[idx_v], add=True)

        return carry

    lax.fori_loop(0, GRND, round_, 0)
    plsc.subcore_barrier()

    def wchunk(k, carry):
        c = sid + k * NS

        @pl.when(c < NWCH)
        def _():
            pltpu.sync_copy(agg_sh.at[pl.ds(c * WCH, WCH)],
                            pcat.at[pl.ds(cid * N + c * WCH, WCH)])

        return carry

    lax.fori_loop(0, WRND, wchunk, 0)


@functools.partial(
    pl.kernel,
    out_type=jax.ShapeDtypeStruct((2 * N, TW), jnp.float32),
    mesh=_MESH,
    scratch_types=[
        pltpu.VMEM((CHG,), jnp.int32),
        pltpu.VMEM((CHG4, TW), jnp.float32),
        pltpu.VMEM((CHG, TW), jnp.float32),
        pltpu.VMEM((ZCH, TW), jnp.float32),
        pltpu.VMEM_SHARED((N, TW), jnp.float32),
        pltpu.SemaphoreType.DMA,
    ],
)
def _scatter(mp, dst, pcat, idx_v, mrows_v, u_v, zbuf_v, agg_sh, sem):
    _scatter_body(mp, dst, pcat, idx_v, mrows_v, u_v, zbuf_v, agg_sh, sem)


def _h0_body(x_ref, w_ref, b_ref, o_ref):
    h = (jnp.dot(x_ref[...], w_ref[...], preferred_element_type=jnp.float32)
         + b_ref[...])
    pad = jnp.concatenate([h, jnp.zeros((N, TW - DOUT), jnp.float32)], axis=1)
    o_ref[...] = jnp.concatenate([pad, jnp.zeros((N, TW), jnp.float32)],
                                 axis=0)


def _h0(node_feats, proj_W, proj_b, nn_bias):
    return pl.pallas_call(
        _h0_body,
        out_shape=jax.ShapeDtypeStruct((2 * N, TW), jnp.float32),
    )(node_feats, proj_W, (proj_b - nn_bias).reshape(1, DOUT))


def _we_body(efp_ref, w1_ref, b1_ref, w2_ref, b2_ref, o_ref):
    outs = []
    for k in range(4):
        z = jax.nn.relu(
            jnp.dot(efp_ref[:, k * DE:(k + 1) * DE], w1_ref[...],
                    preferred_element_type=jnp.float32) + b1_ref[...])
        w = jnp.dot(z, w2_ref[...], preferred_element_type=jnp.float32) \
            + b2_ref[...]
        outs.append(w.astype(jnp.bfloat16))
    o_ref[...] = jnp.concatenate(outs, axis=1)


def _we(efp, W1, b1, W2, b2):
    grid = E4 // (EB_W // 4)
    eb4 = EB_W // 4
    return pl.pallas_call(
        _we_body,
        grid=(grid,),
        in_specs=[
            pl.BlockSpec((eb4, 4 * DE), lambda i: (i, 0)),
            pl.BlockSpec((DE, DH), lambda i: (0, 0)),
            pl.BlockSpec((1, DH), lambda i: (0, 0)),
            pl.BlockSpec((DH, DOUT * DOUT), lambda i: (0, 0)),
            pl.BlockSpec((1, DOUT * DOUT), lambda i: (0, 0)),
        ],
        out_specs=pl.BlockSpec((eb4, 4 * DOUT * DOUT), lambda i: (i, 0)),
        out_shape=jax.ShapeDtypeStruct((E4, 4 * DOUT * DOUT), jnp.bfloat16),
    )(efp, W1, b1.reshape(1, DH), W2, b2.reshape(1, DOUT * DOUT))


def _msg_body(hsp0_ref, hsp1_ref, bt_ref, wep_ref, rp_ref, o_ref):
    eb4 = EB_M // 4
    hp = jax.nn.relu(hsp0_ref[...] + hsp1_ref[...] + bt_ref[...])
    mks = []
    for k in range(4):
        hE = jnp.dot(hp, rp_ref[:, k * 1024:(k + 1) * 1024],
                     preferred_element_type=jnp.float32)
        prod = wep_ref[:, k * 1024:(k + 1) * 1024].astype(jnp.float32) * hE
        s = jnp.zeros((eb4, 128), jnp.float32)
        for c in range(8):
            s = s + prod[:, c * 128:(c + 1) * 128]
        mks.append((s[:, 0:32] + s[:, 32:64]) + (s[:, 64:96] + s[:, 96:128]))
    o_ref[...] = jnp.concatenate(mks, axis=1)


def _rp_mat():
    r = np.zeros((TW, 4 * DOUT * DOUT), np.float32)
    for k in range(4):
        for i in range(DOUT):
            r[k * DOUT + i,
              k * DOUT * DOUT + i * DOUT:k * DOUT * DOUT + (i + 1) * DOUT] = 1.0
    return jnp.asarray(r)


def _messages(hsp0, hsp1, nn_bias, wep):
    eb4 = EB_M // 4
    grid = E4 // eb4
    bt = jnp.tile(nn_bias, 4).reshape(1, TW)
    return pl.pallas_call(
        _msg_body,
        grid=(grid,),
        in_specs=[
            pl.BlockSpec((eb4, TW), lambda i: (i, 0)),
            pl.BlockSpec((eb4, TW), lambda i: (i, 0)),
            pl.BlockSpec((1, TW), lambda i: (0, 0)),
            pl.BlockSpec((eb4, 4 * DOUT * DOUT), lambda i: (i, 0)),
            pl.BlockSpec((TW, 4 * DOUT * DOUT), lambda i: (0, 0)),
        ],
        out_specs=pl.BlockSpec((eb4, TW), lambda i: (i, 0)),
        out_shape=jax.ShapeDtypeStruct((E4, TW), jnp.float32),
    )(hsp0, hsp1, bt, wep, _rp_mat())


def _final_body(pc_ref, b_ref, w_ref, g_ref, bb_ref, atom_ref, p_ref):
    h = jax.nn.relu(pc_ref[0:N, 0:DOUT] + pc_ref[N:2 * N, 0:DOUT] + b_ref[...])
    mu = jnp.mean(h, axis=-1, keepdims=True)
    var = jnp.mean((h - mu) ** 2, axis=-1, keepdims=True)
    atom_ref[...] = (h - mu) / jnp.sqrt(var + 1e-5) * g_ref[...] + bb_ref[...]
    p = jnp.dot(h, w_ref[...], preferred_element_type=jnp.float32)
    zpad = jnp.zeros((N, TW - 2 * DOUT), jnp.float32)
    top = jnp.concatenate([p[:, 0:DOUT], jnp.zeros((N, DOUT), jnp.float32),
                           zpad], axis=1)
    bot = jnp.concatenate([jnp.zeros((N, DOUT), jnp.float32), p[:, DOUT:],
                           zpad], axis=1)
    p_ref[...] = jnp.concatenate([top, bot], axis=0)


def _final(pcat, nn_bias, Wb, gamma, beta):
    wcat = jnp.concatenate([Wb[:DOUT], Wb[DOUT:]], axis=1)
    return pl.pallas_call(
        _final_body,
        out_shape=[jax.ShapeDtypeStruct((N, DOUT), jnp.float32),
                   jax.ShapeDtypeStruct((2 * N, TW), jnp.float32)],
    )(pcat, nn_bias.reshape(1, DOUT), wcat,
      gamma.reshape(1, DOUT), beta.reshape(1, DOUT))


def _bond_body(x_ref, y_ref, bb_ref, g_ref, b_ref, o_ref):
    x = x_ref[:, 0:DOUT] + y_ref[:, DOUT:2 * DOUT] + bb_ref[...]
    mu = jnp.mean(x, axis=-1, keepdims=True)
    var = jnp.mean((x - mu) ** 2, axis=-1, keepdims=True)
    o_ref[...] = (x - mu) / jnp.sqrt(var + 1e-5) * g_ref[...] + b_ref[...]


def _bond(bpre0, bpre1, bb, gamma, beta):
    grid = E // EB_L
    return pl.pallas_call(
        _bond_body,
        grid=(grid,),
        in_specs=[
            pl.BlockSpec((EB_L, TW), lambda i: (i, 0)),
            pl.BlockSpec((EB_L, TW), lambda i: (i, 0)),
            pl.BlockSpec((1, DOUT), lambda i: (0, 0)),
            pl.BlockSpec((1, DOUT), lambda i: (0, 0)),
            pl.BlockSpec((1, DOUT), lambda i: (0, 0)),
        ],
        out_specs=pl.BlockSpec((EB_L, DOUT), lambda i: (i, 0)),
        out_shape=jax.ShapeDtypeStruct((E, DOUT), jnp.float32),
    )(bpre0, bpre1, bb.reshape(1, DOUT), gamma.reshape(1, DOUT),
      beta.reshape(1, DOUT))


def kernel(node_feats, edge_feats, edge_index, proj_W, proj_b, W1, b1, W2, b2,
           nn_bias, Wb, bb, gamma, beta):
    src = edge_index[0]
    dst = edge_index[1]
    srcb = src + N
    dstb = dst + N
    efp = edge_feats.reshape(E4, 4 * DE)

    tab = _h0(node_feats, proj_W, proj_b, nn_bias)
    wep = _we(efp, W1, b1, W2, b2)

    for _ in range(STEPS):
        hsp0, hsp1 = _gather_packed(tab, src, srcb)
        mp = _messages(hsp0, hsp1, nn_bias, wep)
        tab = _scatter(mp, dst)

    atom, btab = _final(tab, nn_bias, Wb, gamma, beta)
    bpre0, bpre1 = _gather_wide(btab, src, dstb)
    bond = _bond(bpre0, bpre1, bb, gamma, beta)
    return (atom, bond)


# speedup vs baseline: 3.0448x; 1.1527x over previous
import functools

import jax
import jax.numpy as jnp
import numpy as np
from jax import lax
from jax.experimental import pallas as pl
from jax.experimental.pallas import tpu as pltpu
from jax.experimental.pallas import tpu_sc as plsc

N = 10000
E = 160000
E4 = E // 4
DIN = 128
DE = 16
DOUT = 32
DH = 128
STEPS = 6
TW = 128

NC = 2
NS = 16
NW = NC * NS

CHG = 256
CHG4 = CHG // 4
NCHG = E // CHG
GRND = -(-NCHG // NW)

WCH = 200
NWCH = N // WCH
WRND = -(-NWCH // NS)
ZCH = 40
NZC = N // ZCH
NZR = -(-NZC // NS)

EB_W = 1600
EB_M = 1600
EB_L = 2000

_MESH = plsc.VectorSubcoreMesh(core_axis_name="c", subcore_axis_name="s")


def _gather_packed_body(tab, idx, out, idx_v, rows_v, pk_v, sem0):
    wid = lax.axis_index("s") * NC + lax.axis_index("c")

    def round_(r, carry):
        c = wid + r * NW

        @pl.when(c < NCHG)
        def _():
            b = c * CHG
            pltpu.sync_copy(idx.at[pl.ds(b, CHG)], idx_v)
            pltpu.async_copy(tab.at[idx_v], rows_v, sem0).wait()

            def pack(g, carry2):
                for k in range(4):
                    for h in range(2):
                        pk_v[g, pl.ds(k * 32 + h * 16, 16)] = (
                            rows_v[4 * g + k, pl.ds(h * 16, 16)])
                return carry2

            lax.fori_loop(0, CHG4, pack, 0)
            pltpu.sync_copy(pk_v, out.at[pl.ds(c * CHG4, CHG4)])

        return carry

    lax.fori_loop(0, GRND, round_, 0)


@functools.partial(
    pl.kernel,
    out_type=jax.ShapeDtypeStruct((E4, TW), jnp.float32),
    mesh=_MESH,
    scratch_types=[
        pltpu.VMEM((CHG,), jnp.int32),
        pltpu.VMEM((CHG, TW), jnp.float32),
        pltpu.VMEM((CHG4, TW), jnp.float32),
        pltpu.SemaphoreType.DMA,
    ],
)
def _gather_packed(tab, idx, out, idx_v, rows_v, pk_v, sem0):
    _gather_packed_body(tab, idx, out, idx_v, rows_v, pk_v, sem0)


def _gather_wide_body(tab, idxa, idxb, outa, outb,
                      idx_va, idx_vb, rows_va, rows_vb, sem0, sem1):
    wid = lax.axis_index("s") * NC + lax.axis_index("c")

    def round_(r, carry):
        c = wid + r * NW

        @pl.when(c < NCHG)
        def _():
            b = c * CHG
            pltpu.sync_copy(idxa.at[pl.ds(b, CHG)], idx_va)
            pltpu.sync_copy(idxb.at[pl.ds(b, CHG)], idx_vb)
            cpa = pltpu.async_copy(tab.at[idx_va], rows_va, sem0)
            cpb = pltpu.async_copy(tab.at[idx_vb], rows_vb, sem1)
            cpa.wait()
            cpb.wait()
            pltpu.sync_copy(rows_va, outa.at[pl.ds(b, CHG)])
            pltpu.sync_copy(rows_vb, outb.at[pl.ds(b, CHG)])

        return carry

    lax.fori_loop(0, GRND, round_, 0)


@functools.partial(
    pl.kernel,
    out_type=[jax.ShapeDtypeStruct((E, TW), jnp.float32),
              jax.ShapeDtypeStruct((E, TW), jnp.float32)],
    mesh=_MESH,
    scratch_types=[
        pltpu.VMEM((CHG,), jnp.int32),
        pltpu.VMEM((CHG,), jnp.int32),
        pltpu.VMEM((CHG, TW), jnp.float32),
        pltpu.VMEM((CHG, TW), jnp.float32),
        pltpu.SemaphoreType.DMA,
        pltpu.SemaphoreType.DMA,
    ],
)
def _gather_wide(tab, idxa, idxb, outa, outb,
                 idx_va, idx_vb, rows_va, rows_vb, sem0, sem1):
    _gather_wide_body(tab, idxa, idxb, outa, outb,
                      idx_va, idx_vb, rows_va, rows_vb, sem0, sem1)


def _scatter_body(mp, dst, pcat, idx_v, mrows_v, u_v, zbuf_v, agg_sh, sem):
    cid = lax.axis_index("c")
    sid = lax.axis_index("s")
    wid = sid * NC + cid

    def zrow(r, carry):
        for c16 in range(TW // 16):
            zbuf_v[r, pl.ds(c16 * 16, 16)] = jnp.zeros((16,), jnp.float32)
        return carry

    lax.fori_loop(0, ZCH, zrow, 0)

    def zchunk(k, carry):
        c = sid + k * NS

        @pl.when(c < NZC)
        def _():
            pltpu.sync_copy(zbuf_v, agg_sh.at[pl.ds(c * ZCH, ZCH)])

        return carry

    lax.fori_loop(0, NZR, zchunk, 0)
    plsc.subcore_barrier()

    def round_(r, carry):
        c = wid + r * NW

        @pl.when(c < NCHG)
        def _():
            pltpu.sync_copy(dst.at[pl.ds(c * CHG, CHG)], idx_v)
            pltpu.sync_copy(mp.at[pl.ds(c * CHG4, CHG4)], mrows_v)

            def unpack(g, carry2):
                for k in range(4):
                    for h in range(2):
                        u_v[4 * g + k, pl.ds(h * 16, 16)] = (
                            mrows_v[g, pl.ds(k * 32 + h * 16, 16)])
                return carry2

            lax.fori_loop(0, CHG4, unpack, 0)
            pltpu.sync_copy(u_v, agg_sh.at[idx_v], add=True)

        return carry

    lax.fori_loop(0, GRND, round_, 0)
    plsc.subcore_barrier()

    def wchunk(k, carry):
        c = sid + k * NS

        @pl.when(c < NWCH)
        def _():
            pltpu.sync_copy(agg_sh.at[pl.ds(c * WCH, WCH)],
                            pcat.at[pl.ds(cid * N + c * WCH, WCH)])

        return carry

    lax.fori_loop(0, WRND, wchunk, 0)


@functools.partial(
    pl.kernel,
    out_type=jax.ShapeDtypeStruct((2 * N, TW), jnp.float32),
    mesh=_MESH,
    scratch_types=[
        pltpu.VMEM((CHG,), jnp.int32),
        pltpu.VMEM((CHG4, TW), jnp.float32),
        pltpu.VMEM((CHG, TW), jnp.float32),
        pltpu.VMEM((ZCH, TW), jnp.float32),
        pltpu.VMEM_SHARED((N, TW), jnp.float32),
        pltpu.SemaphoreType.DMA,
    ],
)
def _scatter(mp, dst, pcat, idx_v, mrows_v, u_v, zbuf_v, agg_sh, sem):
    _scatter_body(mp, dst, pcat, idx_v, mrows_v, u_v, zbuf_v, agg_sh, sem)


def _h0_body(x_ref, w_ref, b_ref, o_ref):
    h = jax.nn.relu(
        jnp.dot(x_ref[...], w_ref[...], preferred_element_type=jnp.float32)
        + b_ref[...])
    o_ref[...] = jnp.concatenate(
        [h, jnp.zeros((N, TW - DOUT), jnp.float32)], axis=1)


def _h0(node_feats, proj_W, proj_b):
    return pl.pallas_call(
        _h0_body,
        out_shape=jax.ShapeDtypeStruct((N, TW), jnp.float32),
    )(node_feats, proj_W, proj_b.reshape(1, DOUT))


def _comb_body(pc_ref, b_ref, o_ref):
    h = jax.nn.relu(pc_ref[0:N, 0:DOUT] + pc_ref[N:2 * N, 0:DOUT] + b_ref[...])
    o_ref[...] = jnp.concatenate(
        [h, jnp.zeros((N, TW - DOUT), jnp.float32)], axis=1)


def _comb(pcat, nn_bias):
    return pl.pallas_call(
        _comb_body,
        out_shape=jax.ShapeDtypeStruct((N, TW), jnp.float32),
    )(pcat, nn_bias.reshape(1, DOUT))


def _we_body(efp_ref, w1_ref, b1_ref, w2_ref, b2_ref, o_ref):
    outs = []
    for k in range(4):
        z = jax.nn.relu(
            jnp.dot(efp_ref[:, k * DE:(k + 1) * DE], w1_ref[...],
                    preferred_element_type=jnp.float32) + b1_ref[...])
        w = jnp.dot(z, w2_ref[...], preferred_element_type=jnp.float32) \
            + b2_ref[...]
        outs.append(w.astype(jnp.bfloat16))
    o_ref[...] = jnp.concatenate(outs, axis=1)


def _we(efp, W1, b1, W2, b2):
    grid = E4 // (EB_W // 4)
    eb4 = EB_W // 4
    return pl.pallas_call(
        _we_body,
        grid=(grid,),
        in_specs=[
            pl.BlockSpec((eb4, 4 * DE), lambda i: (i, 0)),
            pl.BlockSpec((DE, DH), lambda i: (0, 0)),
            pl.BlockSpec((1, DH), lambda i: (0, 0)),
            pl.BlockSpec((DH, DOUT * DOUT), lambda i: (0, 0)),
            pl.BlockSpec((1, DOUT * DOUT), lambda i: (0, 0)),
        ],
        out_specs=pl.BlockSpec((eb4, 4 * DOUT * DOUT), lambda i: (i, 0)),
        out_shape=jax.ShapeDtypeStruct((E4, 4 * DOUT * DOUT), jnp.bfloat16),
    )(efp, W1, b1.reshape(1, DH), W2, b2.reshape(1, DOUT * DOUT))


def _msg_body(hsp_ref, wep_ref, rp_ref, o_ref):
    eb4 = EB_M // 4
    hp = hsp_ref[...]
    mks = []
    for k in range(4):
        hE = jnp.dot(hp, rp_ref[:, k * 1024:(k + 1) * 1024],
                     preferred_element_type=jnp.float32)
        prod = wep_ref[:, k * 1024:(k + 1) * 1024].astype(jnp.float32) * hE
        s = jnp.zeros((eb4, 128), jnp.float32)
        for c in range(8):
            s = s + prod[:, c * 128:(c + 1) * 128]
        mks.append((s[:, 0:32] + s[:, 32:64]) + (s[:, 64:96] + s[:, 96:128]))
    o_ref[...] = jnp.concatenate(mks, axis=1)


def _rp_mat():
    r = np.zeros((TW, 4 * DOUT * DOUT), np.float32)
    for k in range(4):
        for i in range(DOUT):
            r[k * DOUT + i,
              k * DOUT * DOUT + i * DOUT:k * DOUT * DOUT + (i + 1) * DOUT] = 1.0
    return jnp.asarray(r)


def _messages(hsp, wep):
    eb4 = EB_M // 4
    grid = E4 // eb4
    return pl.pallas_call(
        _msg_body,
        grid=(grid,),
        in_specs=[
            pl.BlockSpec((eb4, TW), lambda i: (i, 0)),
            pl.BlockSpec((eb4, 4 * DOUT * DOUT), lambda i: (i, 0)),
            pl.BlockSpec((TW, 4 * DOUT * DOUT), lambda i: (0, 0)),
        ],
        out_specs=pl.BlockSpec((eb4, TW), lambda i: (i, 0)),
        out_shape=jax.ShapeDtypeStruct((E4, TW), jnp.float32),
    )(hsp, wep, _rp_mat())


def _final_body(pc_ref, b_ref, w_ref, g_ref, bb_ref, atom_ref, p_ref):
    h = jax.nn.relu(pc_ref[0:N, 0:DOUT] + pc_ref[N:2 * N, 0:DOUT] + b_ref[...])
    mu = jnp.mean(h, axis=-1, keepdims=True)
    var = jnp.mean((h - mu) ** 2, axis=-1, keepdims=True)
    atom_ref[...] = (h - mu) / jnp.sqrt(var + 1e-5) * g_ref[...] + bb_ref[...]
    p = jnp.dot(h, w_ref[...], preferred_element_type=jnp.float32)
    zpad = jnp.zeros((N, TW - 2 * DOUT), jnp.float32)
    top = jnp.concatenate([p[:, 0:DOUT], jnp.zeros((N, DOUT), jnp.float32),
                           zpad], axis=1)
    bot = jnp.concatenate([jnp.zeros((N, DOUT), jnp.float32), p[:, DOUT:],
                           zpad], axis=1)
    p_ref[...] = jnp.concatenate([top, bot], axis=0)


def _final(pcat, nn_bias, Wb, gamma, beta):
    wcat = jnp.concatenate([Wb[:DOUT], Wb[DOUT:]], axis=1)
    return pl.pallas_call(
        _final_body,
        out_shape=[jax.ShapeDtypeStruct((N, DOUT), jnp.float32),
                   jax.ShapeDtypeStruct((2 * N, TW), jnp.float32)],
    )(pcat, nn_bias.reshape(1, DOUT), wcat,
      gamma.reshape(1, DOUT), beta.reshape(1, DOUT))


def _bond_body(x_ref, y_ref, bb_ref, g_ref, b_ref, o_ref):
    x = x_ref[:, 0:DOUT] + y_ref[:, DOUT:2 * DOUT] + bb_ref[...]
    mu = jnp.mean(x, axis=-1, keepdims=True)
    var = jnp.mean((x - mu) ** 2, axis=-1, keepdims=True)
    o_ref[...] = (x - mu) / jnp.sqrt(var + 1e-5) * g_ref[...] + b_ref[...]


def _bond(bpre0, bpre1, bb, gamma, beta):
    grid = E // EB_L
    return pl.pallas_call(
        _bond_body,
        grid=(grid,),
        in_specs=[
            pl.BlockSpec((EB_L, TW), lambda i: (i, 0)),
            pl.BlockSpec((EB_L, TW), lambda i: (i, 0)),
            pl.BlockSpec((1, DOUT), lambda i: (0, 0)),
            pl.BlockSpec((1, DOUT), lambda i: (0, 0)),
            pl.BlockSpec((1, DOUT), lambda i: (0, 0)),
        ],
        out_specs=pl.BlockSpec((EB_L, DOUT), lambda i: (i, 0)),
        out_shape=jax.ShapeDtypeStruct((E, DOUT), jnp.float32),
    )(bpre0, bpre1, bb.reshape(1, DOUT), gamma.reshape(1, DOUT),
      beta.reshape(1, DOUT))


def kernel(node_feats, edge_feats, edge_index, proj_W, proj_b, W1, b1, W2, b2,
           nn_bias, Wb, bb, gamma, beta):
    src = edge_index[0]
    dst = edge_index[1]
    dstb = dst + N
    efp = edge_feats.reshape(E4, 4 * DE)

    comb = _h0(node_feats, proj_W, proj_b)
    wep = _we(efp, W1, b1, W2, b2)

    pcat = None
    for step in range(STEPS):
        if step > 0:
            comb = _comb(pcat, nn_bias)
        hsp = _gather_packed(comb, src)
        mp = _messages(hsp, wep)
        pcat = _scatter(mp, dst)

    atom, btab = _final(pcat, nn_bias, Wb, gamma, beta)
    bpre0, bpre1 = _gather_wide(btab, src, dstb)
    bond = _bond(bpre0, bpre1, bb, gamma, beta)
    return (atom, bond)


# speedup vs baseline: 3.4853x; 1.1446x over previous
import functools

import jax
import jax.numpy as jnp
import numpy as np
from jax import lax
from jax.experimental import pallas as pl
from jax.experimental.pallas import tpu as pltpu
from jax.experimental.pallas import tpu_sc as plsc

N = 10000
E = 160000
E4 = E // 4
DIN = 128
DE = 16
DOUT = 32
DH = 128
STEPS = 6
TW = 128

NC = 2
NS = 16
NW = NC * NS

CHG = 256
CHG4 = CHG // 4
NCHG = E // CHG
GRND = -(-NCHG // NW)

WCH = 200
NWCH = N // WCH
WRND = -(-NWCH // NS)
ZCH = 40
NZC = N // ZCH
NZR = -(-NZC // NS)

EB_W = 1600
EB_M = 3200
EB_L = 2000

_MESH = plsc.VectorSubcoreMesh(core_axis_name="c", subcore_axis_name="s")


def _gather_packed_body(tab, idx, out, idx_v, rows_v, pk_v, sem0):
    wid = lax.axis_index("s") * NC + lax.axis_index("c")

    def round_(r, carry):
        c = wid + r * NW

        @pl.when(c < NCHG)
        def _():
            b = c * CHG
            pltpu.sync_copy(idx.at[pl.ds(b, CHG)], idx_v)
            pltpu.async_copy(tab.at[idx_v], rows_v, sem0).wait()

            for g in range(CHG4):
                for k in range(4):
                    pk_v[g, pl.ds(k * 32, 32)] = rows_v[4 * g + k,
                                                        pl.ds(0, 32)]
            pltpu.sync_copy(pk_v, out.at[pl.ds(c * CHG4, CHG4)])

        return carry

    lax.fori_loop(0, GRND, round_, 0)


@functools.partial(
    pl.kernel,
    out_type=jax.ShapeDtypeStruct((E4, TW), jnp.float32),
    mesh=_MESH,
    scratch_types=[
        pltpu.VMEM((CHG,), jnp.int32),
        pltpu.VMEM((CHG, TW), jnp.float32),
        pltpu.VMEM((CHG4, TW), jnp.float32),
        pltpu.SemaphoreType.DMA,
    ],
)
def _gather_packed(tab, idx, out, idx_v, rows_v, pk_v, sem0):
    _gather_packed_body(tab, idx, out, idx_v, rows_v, pk_v, sem0)


def _gather_wide_body(tab, idxa, idxb, outa, outb,
                      idx_va, idx_vb, rows_va, rows_vb, sem0, sem1):
    wid = lax.axis_index("s") * NC + lax.axis_index("c")

    def round_(r, carry):
        c = wid + r * NW

        @pl.when(c < NCHG)
        def _():
            b = c * CHG
            pltpu.sync_copy(idxa.at[pl.ds(b, CHG)], idx_va)
            pltpu.sync_copy(idxb.at[pl.ds(b, CHG)], idx_vb)
            cpa = pltpu.async_copy(tab.at[idx_va], rows_va, sem0)
            cpb = pltpu.async_copy(tab.at[idx_vb], rows_vb, sem1)
            cpa.wait()
            cpb.wait()
            pltpu.sync_copy(rows_va, outa.at[pl.ds(b, CHG)])
            pltpu.sync_copy(rows_vb, outb.at[pl.ds(b, CHG)])

        return carry

    lax.fori_loop(0, GRND, round_, 0)


@functools.partial(
    pl.kernel,
    out_type=[jax.ShapeDtypeStruct((E, TW), jnp.float32),
              jax.ShapeDtypeStruct((E, TW), jnp.float32)],
    mesh=_MESH,
    scratch_types=[
        pltpu.VMEM((CHG,), jnp.int32),
        pltpu.VMEM((CHG,), jnp.int32),
        pltpu.VMEM((CHG, TW), jnp.float32),
        pltpu.VMEM((CHG, TW), jnp.float32),
        pltpu.SemaphoreType.DMA,
        pltpu.SemaphoreType.DMA,
    ],
)
def _gather_wide(tab, idxa, idxb, outa, outb,
                 idx_va, idx_vb, rows_va, rows_vb, sem0, sem1):
    _gather_wide_body(tab, idxa, idxb, outa, outb,
                      idx_va, idx_vb, rows_va, rows_vb, sem0, sem1)


def _scatter_body(mp, dst, pcat, idx_v, mrows_v, u_v, zbuf_v, agg_sh, sem):
    cid = lax.axis_index("c")
    sid = lax.axis_index("s")
    wid = sid * NC + cid

    def zrow(r, carry):
        for c16 in range(TW // 16):
            zbuf_v[r, pl.ds(c16 * 16, 16)] = jnp.zeros((16,), jnp.float32)
        return carry

    lax.fori_loop(0, ZCH, zrow, 0)

    def zchunk(k, carry):
        c = sid + k * NS

        @pl.when(c < NZC)
        def _():
            pltpu.sync_copy(zbuf_v, agg_sh.at[pl.ds(c * ZCH, ZCH)])

        return carry

    lax.fori_loop(0, NZR, zchunk, 0)
    plsc.subcore_barrier()

    def round_(r, carry):
        c = wid + r * NW

        @pl.when(c < NCHG)
        def _():
            pltpu.sync_copy(dst.at[pl.ds(c * CHG, CHG)], idx_v)
            pltpu.sync_copy(mp.at[pl.ds(c * CHG4, CHG4)], mrows_v)

            def unpack(g, carry2):
                for k in range(4):
                    for h in range(2):
                        u_v[4 * g + k, pl.ds(h * 16, 16)] = (
                            mrows_v[g, pl.ds(k * 32 + h * 16, 16)])
                return carry2

            lax.fori_loop(0, CHG4, unpack, 0)
            pltpu.sync_copy(u_v, agg_sh.at[idx_v], add=True)

        return carry

    lax.fori_loop(0, GRND, round_, 0)
    plsc.subcore_barrier()

    def wchunk(k, carry):
        c = sid + k * NS

        @pl.when(c < NWCH)
        def _():
            pltpu.sync_copy(agg_sh.at[pl.ds(c * WCH, WCH)],
                            pcat.at[pl.ds(cid * N + c * WCH, WCH)])

        return carry

    lax.fori_loop(0, WRND, wchunk, 0)


@functools.partial(
    pl.kernel,
    out_type=jax.ShapeDtypeStruct((2 * N, TW), jnp.float32),
    mesh=_MESH,
    scratch_types=[
        pltpu.VMEM((CHG,), jnp.int32),
        pltpu.VMEM((CHG4, TW), jnp.float32),
        pltpu.VMEM((CHG, TW), jnp.float32),
        pltpu.VMEM((ZCH, TW), jnp.float32),
        pltpu.VMEM_SHARED((N, TW), jnp.float32),
        pltpu.SemaphoreType.DMA,
    ],
)
def _scatter(mp, dst, pcat, idx_v, mrows_v, u_v, zbuf_v, agg_sh, sem):
    _scatter_body(mp, dst, pcat, idx_v, mrows_v, u_v, zbuf_v, agg_sh, sem)


def _h0_body(x_ref, w_ref, b_ref, o_ref):
    h = jax.nn.relu(
        jnp.dot(x_ref[...], w_ref[...], preferred_element_type=jnp.float32)
        + b_ref[...])
    o_ref[...] = jnp.concatenate(
        [h, jnp.zeros((N, TW - DOUT), jnp.float32)], axis=1)


def _h0(node_feats, proj_W, proj_b):
    return pl.pallas_call(
        _h0_body,
        out_shape=jax.ShapeDtypeStruct((N, TW), jnp.float32),
    )(node_feats, proj_W, proj_b.reshape(1, DOUT))


def _comb_body(pc_ref, b_ref, o_ref):
    h = jax.nn.relu(pc_ref[0:N, 0:DOUT] + pc_ref[N:2 * N, 0:DOUT] + b_ref[...])
    o_ref[...] = jnp.concatenate(
        [h, jnp.zeros((N, TW - DOUT), jnp.float32)], axis=1)


def _comb(pcat, nn_bias):
    return pl.pallas_call(
        _comb_body,
        out_shape=jax.ShapeDtypeStruct((N, TW), jnp.float32),
    )(pcat, nn_bias.reshape(1, DOUT))


def _we_body(efp_ref, w1_ref, b1_ref, w2_ref, b2_ref, o_ref):
    outs = []
    for k in range(4):
        z = jax.nn.relu(
            jnp.dot(efp_ref[:, k * DE:(k + 1) * DE], w1_ref[...],
                    preferred_element_type=jnp.float32) + b1_ref[...])
        w = jnp.dot(z, w2_ref[...], preferred_element_type=jnp.float32) \
            + b2_ref[...]
        outs.append(w.astype(jnp.bfloat16))
    o_ref[...] = jnp.concatenate(outs, axis=1)


def _we(efp, W1, b1, W2, b2):
    grid = E4 // (EB_W // 4)
    eb4 = EB_W // 4
    return pl.pallas_call(
        _we_body,
        grid=(grid,),
        in_specs=[
            pl.BlockSpec((eb4, 4 * DE), lambda i: (i, 0)),
            pl.BlockSpec((DE, DH), lambda i: (0, 0)),
            pl.BlockSpec((1, DH), lambda i: (0, 0)),
            pl.BlockSpec((DH, DOUT * DOUT), lambda i: (0, 0)),
            pl.BlockSpec((1, DOUT * DOUT), lambda i: (0, 0)),
        ],
        out_specs=pl.BlockSpec((eb4, 4 * DOUT * DOUT), lambda i: (i, 0)),
        out_shape=jax.ShapeDtypeStruct((E4, 4 * DOUT * DOUT), jnp.bfloat16),
    )(efp, W1, b1.reshape(1, DH), W2, b2.reshape(1, DOUT * DOUT))


def _msg_body(hsp_ref, wep_ref, rp_ref, o_ref):
    eb4 = EB_M // 4
    hp = hsp_ref[...]
    mks = []
    for k in range(4):
        hE = jnp.dot(hp, rp_ref[:, k * 1024:(k + 1) * 1024],
                     preferred_element_type=jnp.float32)
        prod = wep_ref[:, k * 1024:(k + 1) * 1024].astype(jnp.float32) * hE
        s = jnp.zeros((eb4, 128), jnp.float32)
        for c in range(8):
            s = s + prod[:, c * 128:(c + 1) * 128]
        mks.append((s[:, 0:32] + s[:, 32:64]) + (s[:, 64:96] + s[:, 96:128]))
    o_ref[...] = jnp.concatenate(mks, axis=1)


def _rp_mat():
    r = np.zeros((TW, 4 * DOUT * DOUT), np.float32)
    for k in range(4):
        for i in range(DOUT):
            r[k * DOUT + i,
              k * DOUT * DOUT + i * DOUT:k * DOUT * DOUT + (i + 1) * DOUT] = 1.0
    return jnp.asarray(r)


def _messages(hsp, wep):
    eb4 = EB_M // 4
    grid = E4 // eb4
    return pl.pallas_call(
        _msg_body,
        grid=(grid,),
        in_specs=[
            pl.BlockSpec((eb4, TW), lambda i: (i, 0)),
            pl.BlockSpec((eb4, 4 * DOUT * DOUT), lambda i: (i, 0)),
            pl.BlockSpec((TW, 4 * DOUT * DOUT), lambda i: (0, 0)),
        ],
        out_specs=pl.BlockSpec((eb4, TW), lambda i: (i, 0)),
        out_shape=jax.ShapeDtypeStruct((E4, TW), jnp.float32),
    )(hsp, wep, _rp_mat())


def _final_body(pc_ref, b_ref, w_ref, g_ref, bb_ref, atom_ref, p_ref):
    h = jax.nn.relu(pc_ref[0:N, 0:DOUT] + pc_ref[N:2 * N, 0:DOUT] + b_ref[...])
    mu = jnp.mean(h, axis=-1, keepdims=True)
    var = jnp.mean((h - mu) ** 2, axis=-1, keepdims=True)
    atom_ref[...] = (h - mu) / jnp.sqrt(var + 1e-5) * g_ref[...] + bb_ref[...]
    p = jnp.dot(h, w_ref[...], preferred_element_type=jnp.float32)
    zpad = jnp.zeros((N, TW - 2 * DOUT), jnp.float32)
    top = jnp.concatenate([p[:, 0:DOUT], jnp.zeros((N, DOUT), jnp.float32),
                           zpad], axis=1)
    bot = jnp.concatenate([jnp.zeros((N, DOUT), jnp.float32), p[:, DOUT:],
                           zpad], axis=1)
    p_ref[...] = jnp.concatenate([top, bot], axis=0)


def _final(pcat, nn_bias, Wb, gamma, beta):
    wcat = jnp.concatenate([Wb[:DOUT], Wb[DOUT:]], axis=1)
    return pl.pallas_call(
        _final_body,
        out_shape=[jax.ShapeDtypeStruct((N, DOUT), jnp.float32),
                   jax.ShapeDtypeStruct((2 * N, TW), jnp.float32)],
    )(pcat, nn_bias.reshape(1, DOUT), wcat,
      gamma.reshape(1, DOUT), beta.reshape(1, DOUT))


def _bond_body(x_ref, y_ref, bb_ref, g_ref, b_ref, o_ref):
    x = x_ref[:, 0:DOUT] + y_ref[:, DOUT:2 * DOUT] + bb_ref[...]
    mu = jnp.mean(x, axis=-1, keepdims=True)
    var = jnp.mean((x - mu) ** 2, axis=-1, keepdims=True)
    o_ref[...] = (x - mu) / jnp.sqrt(var + 1e-5) * g_ref[...] + b_ref[...]


def _bond(bpre0, bpre1, bb, gamma, beta):
    grid = E // EB_L
    return pl.pallas_call(
        _bond_body,
        grid=(grid,),
        in_specs=[
            pl.BlockSpec((EB_L, TW), lambda i: (i, 0)),
            pl.BlockSpec((EB_L, TW), lambda i: (i, 0)),
            pl.BlockSpec((1, DOUT), lambda i: (0, 0)),
            pl.BlockSpec((1, DOUT), lambda i: (0, 0)),
            pl.BlockSpec((1, DOUT), lambda i: (0, 0)),
        ],
        out_specs=pl.BlockSpec((EB_L, DOUT), lambda i: (i, 0)),
        out_shape=jax.ShapeDtypeStruct((E, DOUT), jnp.float32),
    )(bpre0, bpre1, bb.reshape(1, DOUT), gamma.reshape(1, DOUT),
      beta.reshape(1, DOUT))


def kernel(node_feats, edge_feats, edge_index, proj_W, proj_b, W1, b1, W2, b2,
           nn_bias, Wb, bb, gamma, beta):
    src = edge_index[0]
    dst = edge_index[1]
    dstb = dst + N
    efp = edge_feats.reshape(E4, 4 * DE)

    comb = _h0(node_feats, proj_W, proj_b)
    wep = _we(efp, W1, b1, W2, b2)

    pcat = None
    for step in range(STEPS):
        if step > 0:
            comb = _comb(pcat, nn_bias)
        hsp = _gather_packed(comb, src)
        mp = _messages(hsp, wep)
        pcat = _scatter(mp, dst)

    atom, btab = _final(pcat, nn_bias, Wb, gamma, beta)
    bpre0, bpre1 = _gather_wide(btab, src, dstb)
    bond = _bond(bpre0, bpre1, bb, gamma, beta)
    return (atom, bond)


# speedup vs baseline: 3.7557x; 1.0776x over previous
import functools

import jax
import jax.numpy as jnp
import numpy as np
from jax import lax
from jax.experimental import pallas as pl
from jax.experimental.pallas import tpu as pltpu
from jax.experimental.pallas import tpu_sc as plsc

N = 10000
E = 160000
E4 = E // 4
DIN = 128
DE = 16
DOUT = 32
DH = 128
STEPS = 6
TW = 128

NC = 2
NS = 16
NW = NC * NS

CHG = 256
CHG4 = CHG // 4
NCHG = E // CHG
GRND = -(-NCHG // NW)

WCH = 200
NWCH = N // WCH
WRND = -(-NWCH // NS)
ZCH = 40
NZC = N // ZCH
NZR = -(-NZC // NS)

EB_W = 1600
EB_M = 3200
EB_L = 2000

_MESH = plsc.VectorSubcoreMesh(core_axis_name="c", subcore_axis_name="s")


def _gather_packed_body(tab, idx, out, idx_v, rows_v, pk_v, sem0):
    wid = lax.axis_index("s") * NC + lax.axis_index("c")

    def round_(r, carry):
        c = wid + r * NW

        @pl.when(c < NCHG)
        def _():
            b = c * CHG
            pltpu.sync_copy(idx.at[pl.ds(b, CHG)], idx_v)
            pltpu.async_copy(tab.at[idx_v], rows_v, sem0).wait()

            for g in range(CHG4):
                for k in range(4):
                    pk_v[g, pl.ds(k * 32, 32)] = rows_v[4 * g + k,
                                                        pl.ds(0, 32)]
            pltpu.sync_copy(pk_v, out.at[pl.ds(c * CHG4, CHG4)])

        return carry

    lax.fori_loop(0, GRND, round_, 0)


@functools.partial(
    pl.kernel,
    out_type=jax.ShapeDtypeStruct((E4, TW), jnp.float32),
    mesh=_MESH,
    scratch_types=[
        pltpu.VMEM((CHG,), jnp.int32),
        pltpu.VMEM((CHG, TW), jnp.float32),
        pltpu.VMEM((CHG4, TW), jnp.float32),
        pltpu.SemaphoreType.DMA,
    ],
)
def _gather_packed(tab, idx, out, idx_v, rows_v, pk_v, sem0):
    _gather_packed_body(tab, idx, out, idx_v, rows_v, pk_v, sem0)


def _gather_wide_body(tab, idxa, idxb, outa, outb,
                      idx_va, idx_vb, rows_va, rows_vb, sem0, sem1):
    wid = lax.axis_index("s") * NC + lax.axis_index("c")

    def round_(r, carry):
        c = wid + r * NW

        @pl.when(c < NCHG)
        def _():
            b = c * CHG
            pltpu.sync_copy(idxa.at[pl.ds(b, CHG)], idx_va)
            pltpu.sync_copy(idxb.at[pl.ds(b, CHG)], idx_vb)
            cpa = pltpu.async_copy(tab.at[idx_va], rows_va, sem0)
            cpb = pltpu.async_copy(tab.at[idx_vb], rows_vb, sem1)
            cpa.wait()
            cpb.wait()
            pltpu.sync_copy(rows_va, outa.at[pl.ds(b, CHG)])
            pltpu.sync_copy(rows_vb, outb.at[pl.ds(b, CHG)])

        return carry

    lax.fori_loop(0, GRND, round_, 0)


@functools.partial(
    pl.kernel,
    out_type=[jax.ShapeDtypeStruct((E, TW), jnp.float32),
              jax.ShapeDtypeStruct((E, TW), jnp.float32)],
    mesh=_MESH,
    scratch_types=[
        pltpu.VMEM((CHG,), jnp.int32),
        pltpu.VMEM((CHG,), jnp.int32),
        pltpu.VMEM((CHG, TW), jnp.float32),
        pltpu.VMEM((CHG, TW), jnp.float32),
        pltpu.SemaphoreType.DMA,
        pltpu.SemaphoreType.DMA,
    ],
)
def _gather_wide(tab, idxa, idxb, outa, outb,
                 idx_va, idx_vb, rows_va, rows_vb, sem0, sem1):
    _gather_wide_body(tab, idxa, idxb, outa, outb,
                      idx_va, idx_vb, rows_va, rows_vb, sem0, sem1)


def _scatter_body(mp, dst, pcat, idx_v, mrows_v, u_v, zbuf_v, agg_sh, sem):
    cid = lax.axis_index("c")
    sid = lax.axis_index("s")
    wid = sid * NC + cid

    def zrow(r, carry):
        for c16 in range(TW // 16):
            zbuf_v[r, pl.ds(c16 * 16, 16)] = jnp.zeros((16,), jnp.float32)
        return carry

    lax.fori_loop(0, ZCH, zrow, 0)

    def zchunk(k, carry):
        c = sid + k * NS

        @pl.when(c < NZC)
        def _():
            pltpu.sync_copy(zbuf_v, agg_sh.at[pl.ds(c * ZCH, ZCH)])

        return carry

    lax.fori_loop(0, NZR, zchunk, 0)
    plsc.subcore_barrier()

    def round_(r, carry):
        c = wid + r * NW

        @pl.when(c < NCHG)
        def _():
            pltpu.sync_copy(dst.at[pl.ds(c * CHG, CHG)], idx_v)
            pltpu.sync_copy(mp.at[pl.ds(c * CHG4, CHG4)], mrows_v)

            for g in range(CHG4):
                for k in range(4):
                    for h in range(2):
                        u_v[4 * g + k, pl.ds(h * 16, 16)] = (
                            mrows_v[g, pl.ds(k * 32 + h * 16, 16)])
            pltpu.sync_copy(u_v, agg_sh.at[idx_v], add=True)

        return carry

    lax.fori_loop(0, GRND, round_, 0)
    plsc.subcore_barrier()

    def wchunk(k, carry):
        c = sid + k * NS

        @pl.when(c < NWCH)
        def _():
            pltpu.sync_copy(agg_sh.at[pl.ds(c * WCH, WCH)],
                            pcat.at[pl.ds(cid * N + c * WCH, WCH)])

        return carry

    lax.fori_loop(0, WRND, wchunk, 0)


@functools.partial(
    pl.kernel,
    out_type=jax.ShapeDtypeStruct((2 * N, TW), jnp.float32),
    mesh=_MESH,
    scratch_types=[
        pltpu.VMEM((CHG,), jnp.int32),
        pltpu.VMEM((CHG4, TW), jnp.float32),
        pltpu.VMEM((CHG, TW), jnp.float32),
        pltpu.VMEM((ZCH, TW), jnp.float32),
        pltpu.VMEM_SHARED((N, TW), jnp.float32),
        pltpu.SemaphoreType.DMA,
    ],
)
def _scatter(mp, dst, pcat, idx_v, mrows_v, u_v, zbuf_v, agg_sh, sem):
    _scatter_body(mp, dst, pcat, idx_v, mrows_v, u_v, zbuf_v, agg_sh, sem)


def _h0_body(x_ref, w_ref, b_ref, o_ref):
    h = jax.nn.relu(
        jnp.dot(x_ref[...], w_ref[...], preferred_element_type=jnp.float32)
        + b_ref[...])
    o_ref[...] = jnp.concatenate(
        [h, jnp.zeros((N, TW - DOUT), jnp.float32)], axis=1)


def _h0(node_feats, proj_W, proj_b):
    return pl.pallas_call(
        _h0_body,
        out_shape=jax.ShapeDtypeStruct((N, TW), jnp.float32),
    )(node_feats, proj_W, proj_b.reshape(1, DOUT))


def _comb_body(pc_ref, b_ref, o_ref):
    h = jax.nn.relu(pc_ref[0:N, 0:DOUT] + pc_ref[N:2 * N, 0:DOUT] + b_ref[...])
    o_ref[...] = jnp.concatenate(
        [h, jnp.zeros((N, TW - DOUT), jnp.float32)], axis=1)


def _comb(pcat, nn_bias):
    return pl.pallas_call(
        _comb_body,
        out_shape=jax.ShapeDtypeStruct((N, TW), jnp.float32),
    )(pcat, nn_bias.reshape(1, DOUT))


def _we_body(efp_ref, w1_ref, b1_ref, w2_ref, b2_ref, o_ref):
    outs = []
    for k in range(4):
        z = jax.nn.relu(
            jnp.dot(efp_ref[:, k * DE:(k + 1) * DE], w1_ref[...],
                    preferred_element_type=jnp.float32) + b1_ref[...])
        w = jnp.dot(z, w2_ref[...], preferred_element_type=jnp.float32) \
            + b2_ref[...]
        outs.append(w.astype(jnp.bfloat16))
    o_ref[...] = jnp.concatenate(outs, axis=1)


def _we(efp, W1, b1, W2, b2):
    grid = E4 // (EB_W // 4)
    eb4 = EB_W // 4
    return pl.pallas_call(
        _we_body,
        grid=(grid,),
        in_specs=[
            pl.BlockSpec((eb4, 4 * DE), lambda i: (i, 0)),
            pl.BlockSpec((DE, DH), lambda i: (0, 0)),
            pl.BlockSpec((1, DH), lambda i: (0, 0)),
            pl.BlockSpec((DH, DOUT * DOUT), lambda i: (0, 0)),
            pl.BlockSpec((1, DOUT * DOUT), lambda i: (0, 0)),
        ],
        out_specs=pl.BlockSpec((eb4, 4 * DOUT * DOUT), lambda i: (i, 0)),
        out_shape=jax.ShapeDtypeStruct((E4, 4 * DOUT * DOUT), jnp.bfloat16),
    )(efp, W1, b1.reshape(1, DH), W2, b2.reshape(1, DOUT * DOUT))


def _msg_body(hsp_ref, wep_ref, rp_ref, o_ref):
    eb4 = EB_M // 4
    hp = hsp_ref[...]
    mks = []
    for k in range(4):
        hE = jnp.dot(hp, rp_ref[:, k * 1024:(k + 1) * 1024],
                     preferred_element_type=jnp.float32)
        prod = wep_ref[:, k * 1024:(k + 1) * 1024].astype(jnp.float32) * hE
        s = jnp.zeros((eb4, 128), jnp.float32)
        for c in range(8):
            s = s + prod[:, c * 128:(c + 1) * 128]
        mks.append((s[:, 0:32] + s[:, 32:64]) + (s[:, 64:96] + s[:, 96:128]))
    o_ref[...] = jnp.concatenate(mks, axis=1)


def _rp_mat():
    r = np.zeros((TW, 4 * DOUT * DOUT), np.float32)
    for k in range(4):
        for i in range(DOUT):
            r[k * DOUT + i,
              k * DOUT * DOUT + i * DOUT:k * DOUT * DOUT + (i + 1) * DOUT] = 1.0
    return jnp.asarray(r)


def _messages(hsp, wep):
    eb4 = EB_M // 4
    grid = E4 // eb4
    return pl.pallas_call(
        _msg_body,
        grid=(grid,),
        in_specs=[
            pl.BlockSpec((eb4, TW), lambda i: (i, 0)),
            pl.BlockSpec((eb4, 4 * DOUT * DOUT), lambda i: (i, 0)),
            pl.BlockSpec((TW, 4 * DOUT * DOUT), lambda i: (0, 0)),
        ],
        out_specs=pl.BlockSpec((eb4, TW), lambda i: (i, 0)),
        out_shape=jax.ShapeDtypeStruct((E4, TW), jnp.float32),
    )(hsp, wep, _rp_mat())


def _final_body(pc_ref, b_ref, w_ref, g_ref, bb_ref, atom_ref, p_ref):
    h = jax.nn.relu(pc_ref[0:N, 0:DOUT] + pc_ref[N:2 * N, 0:DOUT] + b_ref[...])
    mu = jnp.mean(h, axis=-1, keepdims=True)
    var = jnp.mean((h - mu) ** 2, axis=-1, keepdims=True)
    atom_ref[...] = (h - mu) / jnp.sqrt(var + 1e-5) * g_ref[...] + bb_ref[...]
    p = jnp.dot(h, w_ref[...], preferred_element_type=jnp.float32)
    zpad = jnp.zeros((N, TW - 2 * DOUT), jnp.float32)
    top = jnp.concatenate([p[:, 0:DOUT], jnp.zeros((N, DOUT), jnp.float32),
                           zpad], axis=1)
    bot = jnp.concatenate([jnp.zeros((N, DOUT), jnp.float32), p[:, DOUT:],
                           zpad], axis=1)
    p_ref[...] = jnp.concatenate([top, bot], axis=0)


def _final(pcat, nn_bias, Wb, gamma, beta):
    wcat = jnp.concatenate([Wb[:DOUT], Wb[DOUT:]], axis=1)
    return pl.pallas_call(
        _final_body,
        out_shape=[jax.ShapeDtypeStruct((N, DOUT), jnp.float32),
                   jax.ShapeDtypeStruct((2 * N, TW), jnp.float32)],
    )(pcat, nn_bias.reshape(1, DOUT), wcat,
      gamma.reshape(1, DOUT), beta.reshape(1, DOUT))


def _bond_body(x_ref, y_ref, bb_ref, g_ref, b_ref, o_ref):
    x = x_ref[:, 0:DOUT] + y_ref[:, DOUT:2 * DOUT] + bb_ref[...]
    mu = jnp.mean(x, axis=-1, keepdims=True)
    var = jnp.mean((x - mu) ** 2, axis=-1, keepdims=True)
    o_ref[...] = (x - mu) / jnp.sqrt(var + 1e-5) * g_ref[...] + b_ref[...]


def _bond(bpre0, bpre1, bb, gamma, beta):
    grid = E // EB_L
    return pl.pallas_call(
        _bond_body,
        grid=(grid,),
        in_specs=[
            pl.BlockSpec((EB_L, TW), lambda i: (i, 0)),
            pl.BlockSpec((EB_L, TW), lambda i: (i, 0)),
            pl.BlockSpec((1, DOUT), lambda i: (0, 0)),
            pl.BlockSpec((1, DOUT), lambda i: (0, 0)),
            pl.BlockSpec((1, DOUT), lambda i: (0, 0)),
        ],
        out_specs=pl.BlockSpec((EB_L, DOUT), lambda i: (i, 0)),
        out_shape=jax.ShapeDtypeStruct((E, DOUT), jnp.float32),
    )(bpre0, bpre1, bb.reshape(1, DOUT), gamma.reshape(1, DOUT),
      beta.reshape(1, DOUT))


def kernel(node_feats, edge_feats, edge_index, proj_W, proj_b, W1, b1, W2, b2,
           nn_bias, Wb, bb, gamma, beta):
    src = edge_index[0]
    dst = edge_index[1]
    dstb = dst + N
    efp = edge_feats.reshape(E4, 4 * DE)

    comb = _h0(node_feats, proj_W, proj_b)
    wep = _we(efp, W1, b1, W2, b2)

    pcat = None
    for step in range(STEPS):
        if step > 0:
            comb = _comb(pcat, nn_bias)
        hsp = _gather_packed(comb, src)
        mp = _messages(hsp, wep)
        pcat = _scatter(mp, dst)

    atom, btab = _final(pcat, nn_bias, Wb, gamma, beta)
    bpre0, bpre1 = _gather_wide(btab, src, dstb)
    bond = _bond(bpre0, bpre1, bb, gamma, beta)
    return (atom, bond)


# speedup vs baseline: 3.8134x; 1.0154x over previous
import functools

import jax
import jax.numpy as jnp
import numpy as np
from jax import lax
from jax.experimental import pallas as pl
from jax.experimental.pallas import tpu as pltpu
from jax.experimental.pallas import tpu_sc as plsc

N = 10000
E = 160000
E4 = E // 4
DIN = 128
DE = 16
DOUT = 32
DH = 128
STEPS = 6
TW = 128

NC = 2
NS = 16
NW = NC * NS

CHG = 256
CHG4 = CHG // 4
NCHG = E // CHG
GRND = -(-NCHG // NW)

WCH = 200
NWCH = N // WCH
WRND = -(-NWCH // NS)
ZCH = 40
NZC = N // ZCH
NZR = -(-NZC // NS)

EB_W = 3200
EB_M = 3200
EB_L = 2000

_MESH = plsc.VectorSubcoreMesh(core_axis_name="c", subcore_axis_name="s")


def _gather_packed_body(tab, idx, out, idx_v, rows_v, pk_v, sem0):
    wid = lax.axis_index("s") * NC + lax.axis_index("c")

    def round_(r, carry):
        c = wid + r * NW

        @pl.when(c < NCHG)
        def _():
            b = c * CHG
            pltpu.sync_copy(idx.at[pl.ds(b, CHG)], idx_v)
            pltpu.async_copy(tab.at[idx_v], rows_v, sem0).wait()

            for g in range(CHG4):
                for k in range(4):
                    pk_v[g, pl.ds(k * 32, 32)] = rows_v[4 * g + k,
                                                        pl.ds(0, 32)]
            pltpu.sync_copy(pk_v, out.at[pl.ds(c * CHG4, CHG4)])

        return carry

    lax.fori_loop(0, GRND, round_, 0)


@functools.partial(
    pl.kernel,
    out_type=jax.ShapeDtypeStruct((E4, TW), jnp.float32),
    mesh=_MESH,
    scratch_types=[
        pltpu.VMEM((CHG,), jnp.int32),
        pltpu.VMEM((CHG, TW), jnp.float32),
        pltpu.VMEM((CHG4, TW), jnp.float32),
        pltpu.SemaphoreType.DMA,
    ],
)
def _gather_packed(tab, idx, out, idx_v, rows_v, pk_v, sem0):
    _gather_packed_body(tab, idx, out, idx_v, rows_v, pk_v, sem0)


def _gather_wide_body(tab, idxa, idxb, outa, outb,
                      idx_va, idx_vb, rows_va, rows_vb, sem0, sem1):
    wid = lax.axis_index("s") * NC + lax.axis_index("c")

    def round_(r, carry):
        c = wid + r * NW

        @pl.when(c < NCHG)
        def _():
            b = c * CHG
            pltpu.sync_copy(idxa.at[pl.ds(b, CHG)], idx_va)
            pltpu.sync_copy(idxb.at[pl.ds(b, CHG)], idx_vb)
            cpa = pltpu.async_copy(tab.at[idx_va], rows_va, sem0)
            cpb = pltpu.async_copy(tab.at[idx_vb], rows_vb, sem1)
            cpa.wait()
            cpb.wait()
            pltpu.sync_copy(rows_va, outa.at[pl.ds(b, CHG)])
            pltpu.sync_copy(rows_vb, outb.at[pl.ds(b, CHG)])

        return carry

    lax.fori_loop(0, GRND, round_, 0)


@functools.partial(
    pl.kernel,
    out_type=[jax.ShapeDtypeStruct((E, TW), jnp.float32),
              jax.ShapeDtypeStruct((E, TW), jnp.float32)],
    mesh=_MESH,
    scratch_types=[
        pltpu.VMEM((CHG,), jnp.int32),
        pltpu.VMEM((CHG,), jnp.int32),
        pltpu.VMEM((CHG, TW), jnp.float32),
        pltpu.VMEM((CHG, TW), jnp.float32),
        pltpu.SemaphoreType.DMA,
        pltpu.SemaphoreType.DMA,
    ],
)
def _gather_wide(tab, idxa, idxb, outa, outb,
                 idx_va, idx_vb, rows_va, rows_vb, sem0, sem1):
    _gather_wide_body(tab, idxa, idxb, outa, outb,
                      idx_va, idx_vb, rows_va, rows_vb, sem0, sem1)


def _scatter_body(mp, dst, pcat, idx_v, mrows_v, u_v, zbuf_v, agg_sh, sem):
    cid = lax.axis_index("c")
    sid = lax.axis_index("s")
    wid = sid * NC + cid

    for r in range(ZCH):
        for c16 in range(TW // 16):
            zbuf_v[r, pl.ds(c16 * 16, 16)] = jnp.zeros((16,), jnp.float32)

    def zchunk(k, carry):
        c = sid + k * NS

        @pl.when(c < NZC)
        def _():
            pltpu.sync_copy(zbuf_v, agg_sh.at[pl.ds(c * ZCH, ZCH)])

        return carry

    lax.fori_loop(0, NZR, zchunk, 0)
    plsc.subcore_barrier()

    def round_(r, carry):
        c = wid + r * NW

        @pl.when(c < NCHG)
        def _():
            pltpu.sync_copy(dst.at[pl.ds(c * CHG, CHG)], idx_v)
            pltpu.sync_copy(mp.at[pl.ds(c * CHG4, CHG4)], mrows_v)

            for g in range(CHG4):
                for k in range(4):
                    for h in range(2):
                        u_v[4 * g + k, pl.ds(h * 16, 16)] = (
                            mrows_v[g, pl.ds(k * 32 + h * 16, 16)])
            pltpu.sync_copy(u_v, agg_sh.at[idx_v], add=True)

        return carry

    lax.fori_loop(0, GRND, round_, 0)
    plsc.subcore_barrier()

    def wchunk(k, carry):
        c = sid + k * NS

        @pl.when(c < NWCH)
        def _():
            pltpu.sync_copy(agg_sh.at[pl.ds(c * WCH, WCH)],
                            pcat.at[pl.ds(cid * N + c * WCH, WCH)])

        return carry

    lax.fori_loop(0, WRND, wchunk, 0)


@functools.partial(
    pl.kernel,
    out_type=jax.ShapeDtypeStruct((2 * N, TW), jnp.float32),
    mesh=_MESH,
    scratch_types=[
        pltpu.VMEM((CHG,), jnp.int32),
        pltpu.VMEM((CHG4, TW), jnp.float32),
        pltpu.VMEM((CHG, TW), jnp.float32),
        pltpu.VMEM((ZCH, TW), jnp.float32),
        pltpu.VMEM_SHARED((N, TW), jnp.float32),
        pltpu.SemaphoreType.DMA,
    ],
)
def _scatter(mp, dst, pcat, idx_v, mrows_v, u_v, zbuf_v, agg_sh, sem):
    _scatter_body(mp, dst, pcat, idx_v, mrows_v, u_v, zbuf_v, agg_sh, sem)


def _h0_body(x_ref, w_ref, b_ref, o_ref):
    h = jax.nn.relu(
        jnp.dot(x_ref[...], w_ref[...], preferred_element_type=jnp.float32)
        + b_ref[...])
    o_ref[...] = jnp.concatenate(
        [h, jnp.zeros((N, TW - DOUT), jnp.float32)], axis=1)


def _h0(node_feats, proj_W, proj_b):
    return pl.pallas_call(
        _h0_body,
        out_shape=jax.ShapeDtypeStruct((N, TW), jnp.float32),
    )(node_feats, proj_W, proj_b.reshape(1, DOUT))


def _comb_body(pc_ref, b_ref, o_ref):
    h = jax.nn.relu(pc_ref[0:N, 0:DOUT] + pc_ref[N:2 * N, 0:DOUT] + b_ref[...])
    o_ref[...] = jnp.concatenate(
        [h, jnp.zeros((N, TW - DOUT), jnp.float32)], axis=1)


def _comb(pcat, nn_bias):
    return pl.pallas_call(
        _comb_body,
        out_shape=jax.ShapeDtypeStruct((N, TW), jnp.float32),
    )(pcat, nn_bias.reshape(1, DOUT))


def _we_body(efp_ref, w1_ref, b1_ref, w2_ref, b2_ref, o_ref):
    outs = []
    for k in range(4):
        z = jax.nn.relu(
            jnp.dot(efp_ref[:, k * DE:(k + 1) * DE], w1_ref[...],
                    preferred_element_type=jnp.float32) + b1_ref[...])
        w = jnp.dot(z, w2_ref[...], preferred_element_type=jnp.float32) \
            + b2_ref[...]
        outs.append(w.astype(jnp.bfloat16))
    o_ref[...] = jnp.concatenate(outs, axis=1)


def _we(efp, W1, b1, W2, b2):
    grid = E4 // (EB_W // 4)
    eb4 = EB_W // 4
    return pl.pallas_call(
        _we_body,
        grid=(grid,),
        in_specs=[
            pl.BlockSpec((eb4, 4 * DE), lambda i: (i, 0)),
            pl.BlockSpec((DE, DH), lambda i: (0, 0)),
            pl.BlockSpec((1, DH), lambda i: (0, 0)),
            pl.BlockSpec((DH, DOUT * DOUT), lambda i: (0, 0)),
            pl.BlockSpec((1, DOUT * DOUT), lambda i: (0, 0)),
        ],
        out_specs=pl.BlockSpec((eb4, 4 * DOUT * DOUT), lambda i: (i, 0)),
        out_shape=jax.ShapeDtypeStruct((E4, 4 * DOUT * DOUT), jnp.bfloat16),
    )(efp, W1, b1.reshape(1, DH), W2, b2.reshape(1, DOUT * DOUT))


def _msg_body(hsp_ref, wep_ref, rp_ref, o_ref):
    eb4 = EB_M // 4
    hp = hsp_ref[...]
    mks = []
    for k in range(4):
        hE = jnp.dot(hp, rp_ref[:, k * 1024:(k + 1) * 1024],
                     preferred_element_type=jnp.float32)
        prod = wep_ref[:, k * 1024:(k + 1) * 1024].astype(jnp.float32) * hE
        s = jnp.zeros((eb4, 128), jnp.float32)
        for c in range(8):
            s = s + prod[:, c * 128:(c + 1) * 128]
        mks.append((s[:, 0:32] + s[:, 32:64]) + (s[:, 64:96] + s[:, 96:128]))
    o_ref[...] = jnp.concatenate(mks, axis=1)


def _rp_mat():
    r = np.zeros((TW, 4 * DOUT * DOUT), np.float32)
    for k in range(4):
        for i in range(DOUT):
            r[k * DOUT + i,
              k * DOUT * DOUT + i * DOUT:k * DOUT * DOUT + (i + 1) * DOUT] = 1.0
    return jnp.asarray(r)


def _messages(hsp, wep):
    eb4 = EB_M // 4
    grid = E4 // eb4
    return pl.pallas_call(
        _msg_body,
        grid=(grid,),
        in_specs=[
            pl.BlockSpec((eb4, TW), lambda i: (i, 0)),
            pl.BlockSpec((eb4, 4 * DOUT * DOUT), lambda i: (i, 0)),
            pl.BlockSpec((TW, 4 * DOUT * DOUT), lambda i: (0, 0)),
        ],
        out_specs=pl.BlockSpec((eb4, TW), lambda i: (i, 0)),
        out_shape=jax.ShapeDtypeStruct((E4, TW), jnp.float32),
    )(hsp, wep, _rp_mat())


def _final_body(pc_ref, b_ref, w_ref, g_ref, bb_ref, atom_ref, p_ref):
    h = jax.nn.relu(pc_ref[0:N, 0:DOUT] + pc_ref[N:2 * N, 0:DOUT] + b_ref[...])
    mu = jnp.mean(h, axis=-1, keepdims=True)
    var = jnp.mean((h - mu) ** 2, axis=-1, keepdims=True)
    atom_ref[...] = (h - mu) / jnp.sqrt(var + 1e-5) * g_ref[...] + bb_ref[...]
    p = jnp.dot(h, w_ref[...], preferred_element_type=jnp.float32)
    zpad = jnp.zeros((N, TW - 2 * DOUT), jnp.float32)
    top = jnp.concatenate([p[:, 0:DOUT], jnp.zeros((N, DOUT), jnp.float32),
                           zpad], axis=1)
    bot = jnp.concatenate([jnp.zeros((N, DOUT), jnp.float32), p[:, DOUT:],
                           zpad], axis=1)
    p_ref[...] = jnp.concatenate([top, bot], axis=0)


def _final(pcat, nn_bias, Wb, gamma, beta):
    wcat = jnp.concatenate([Wb[:DOUT], Wb[DOUT:]], axis=1)
    return pl.pallas_call(
        _final_body,
        out_shape=[jax.ShapeDtypeStruct((N, DOUT), jnp.float32),
                   jax.ShapeDtypeStruct((2 * N, TW), jnp.float32)],
    )(pcat, nn_bias.reshape(1, DOUT), wcat,
      gamma.reshape(1, DOUT), beta.reshape(1, DOUT))


def _bond_body(x_ref, y_ref, bb_ref, g_ref, b_ref, o_ref):
    x = x_ref[:, 0:DOUT] + y_ref[:, DOUT:2 * DOUT] + bb_ref[...]
    mu = jnp.mean(x, axis=-1, keepdims=True)
    var = jnp.mean((x - mu) ** 2, axis=-1, keepdims=True)
    o_ref[...] = (x - mu) / jnp.sqrt(var + 1e-5) * g_ref[...] + b_ref[...]


def _bond(bpre0, bpre1, bb, gamma, beta):
    grid = E // EB_L
    return pl.pallas_call(
        _bond_body,
        grid=(grid,),
        in_specs=[
            pl.BlockSpec((EB_L, TW), lambda i: (i, 0)),
            pl.BlockSpec((EB_L, TW), lambda i: (i, 0)),
            pl.BlockSpec((1, DOUT), lambda i: (0, 0)),
            pl.BlockSpec((1, DOUT), lambda i: (0, 0)),
            pl.BlockSpec((1, DOUT), lambda i: (0, 0)),
        ],
        out_specs=pl.BlockSpec((EB_L, DOUT), lambda i: (i, 0)),
        out_shape=jax.ShapeDtypeStruct((E, DOUT), jnp.float32),
    )(bpre0, bpre1, bb.reshape(1, DOUT), gamma.reshape(1, DOUT),
      beta.reshape(1, DOUT))


def kernel(node_feats, edge_feats, edge_index, proj_W, proj_b, W1, b1, W2, b2,
           nn_bias, Wb, bb, gamma, beta):
    src = edge_index[0]
    dst = edge_index[1]
    dstb = dst + N
    efp = edge_feats.reshape(E4, 4 * DE)

    comb = _h0(node_feats, proj_W, proj_b)
    wep = _we(efp, W1, b1, W2, b2)

    pcat = None
    for step in range(STEPS):
        if step > 0:
            comb = _comb(pcat, nn_bias)
        hsp = _gather_packed(comb, src)
        mp = _messages(hsp, wep)
        pcat = _scatter(mp, dst)

    atom, btab = _final(pcat, nn_bias, Wb, gamma, beta)
    bpre0, bpre1 = _gather_wide(btab, src, dstb)
    bond = _bond(bpre0, bpre1, bb, gamma, beta)
    return (atom, bond)
